# baseline (speedup 1.0000x reference)
import functools

import jax
import jax.numpy as jnp
from jax import lax
from jax.experimental import pallas as pl
from jax.experimental.pallas import tpu as pltpu
from jax.experimental.pallas import tpu_sc as plsc

_WIDTH = 1000
_FEATURE_DIM = 100000
_N = 1024

_INFO = plsc.get_sparse_core_info()
_NW = _INFO.num_cores * _INFO.num_subcores
_ROWS_PER = _N // _NW
_CHUNKS = _ROWS_PER // 16


def _sc_onehot_body(x_hbm, y_hbm, out_hbm, rowbuf, xbuf, ybuf):
    wid = lax.axis_index("s") * _INFO.num_cores + lax.axis_index("c")
    base = wid * _ROWS_PER

    lanes = lax.broadcasted_iota(jnp.int32, (16,), 0)
    ones_f = jnp.ones((16,), jnp.float32)
    zeros_f = jnp.zeros((16,), jnp.float32)

    def zero_body(i, carry):
        rowbuf[pl.ds(pl.multiple_of(i * 16, 16), 16)] = zeros_f
        return carry

    lax.fori_loop(0, _FEATURE_DIM // 16, zero_body, 0, unroll=8)

    for k in range(_CHUNKS):
        cbase = base + k * 16
        pltpu.sync_copy(x_hbm.at[pl.ds(cbase, 16)], xbuf)
        pltpu.sync_copy(y_hbm.at[pl.ds(cbase, 16)], ybuf)
        idxv = xbuf[...] + _WIDTH * ybuf[...]

        def row_body(l, idxv):
            mask = lanes == l
            plsc.store_scatter(rowbuf, [idxv], ones_f, mask=mask)
            pltpu.sync_copy(rowbuf, out_hbm.at[cbase + l])
            plsc.store_scatter(rowbuf, [idxv], zeros_f, mask=mask)
            return idxv

        lax.fori_loop(0, 16, row_body, idxv)


def kernel(state):
    mesh = plsc.VectorSubcoreMesh(core_axis_name="c", subcore_axis_name="s")
    sc_onehot = functools.partial(
        pl.kernel,
        mesh=mesh,
        out_type=jax.ShapeDtypeStruct((_N, _FEATURE_DIM), jnp.float32),
        scratch_types=[
            pltpu.VMEM((_FEATURE_DIM,), jnp.float32),
            pltpu.VMEM((16,), jnp.int32),
            pltpu.VMEM((16,), jnp.int32),
        ],
        compiler_params=pltpu.CompilerParams(needs_layout_passes=False),
    )(_sc_onehot_body)
    return sc_onehot(state[:, 0], state[:, 1])
